# baseline (speedup 1.0000x reference)
import jax
import jax.numpy as jnp
from jax import lax
from jax.experimental import pallas as pl

_BLK = 4000


def _body(gu_ref, fi_ref, out_ref):
    p = gu_ref[...] * fi_ref[...]
    d = lax.broadcasted_iota(jnp.int32, (128, 128), 0)
    c = lax.broadcasted_iota(jnp.int32, (128, 128), 1)
    sel = jnp.where(d // 64 == c, 1.0, 0.0).astype(jnp.float32)
    s = jax.lax.dot_general(
        p, sel, (((1,), (0,)), ((), ())), preferred_element_type=jnp.float32
    )
    out_ref[...] = s[:, :2]


def kernel(gu, fi):
    B, D = gu.shape
    half = B // 2
    g2 = gu.reshape(half, 2 * D)
    f2 = fi.reshape(half, 2 * D)
    grid = half // _BLK
    out = pl.pallas_call(
        _body,
        grid=(grid,),
        in_specs=[
            pl.BlockSpec((_BLK, 2 * D), lambda i: (i, 0)),
            pl.BlockSpec((_BLK, 2 * D), lambda i: (i, 0)),
        ],
        out_specs=pl.BlockSpec((_BLK, 2), lambda i: (i, 0)),
        out_shape=jax.ShapeDtypeStruct((half, 2), jnp.float32),
    )(g2, f2)
    return out.reshape(B)


# speedup vs baseline: 10.0937x; 10.0937x over previous
import jax
import jax.numpy as jnp
from jax.experimental import pallas as pl

_BLKN = 16384


def _body(gu_ref, fi_ref, out_ref):
    p = gu_ref[...] * fi_ref[...]
    out_ref[...] = jnp.sum(p, axis=0)


def kernel(gu, fi):
    B, D = gu.shape
    grid = pl.cdiv(B, _BLKN)
    out = pl.pallas_call(
        _body,
        grid=(grid,),
        in_specs=[
            pl.BlockSpec((D, _BLKN), lambda i: (0, i)),
            pl.BlockSpec((D, _BLKN), lambda i: (0, i)),
        ],
        out_specs=pl.BlockSpec((_BLKN,), lambda i: (i,)),
        out_shape=jax.ShapeDtypeStruct((B,), jnp.float32),
    )(gu.T, fi.T)
    return out
